# baseline (speedup 1.0000x reference)
import functools

import jax
import jax.numpy as jnp
from jax import lax
from jax.experimental import pallas as pl
from jax.experimental.pallas import tpu as pltpu, tpu_sc as plsc

_B = 16384
_D = 64

_NC = 2
_NS = 16
_NW = _NC * _NS
_BPW = _B // _NW

_mesh = plsc.VectorSubcoreMesh(core_axis_name="c", subcore_axis_name="s")


@functools.partial(
    pl.kernel,
    mesh=_mesh,
    out_type=(
        jax.ShapeDtypeStruct((_B, _D), jnp.float32),
        jax.ShapeDtypeStruct((_B, _D), jnp.float32),
        jax.ShapeDtypeStruct((_B, _D), jnp.float32),
    ),
    scratch_types=[
        pltpu.VMEM((_BPW,), jnp.int32),
        pltpu.VMEM((_BPW,), jnp.int32),
        pltpu.VMEM((_BPW,), jnp.int32),
        pltpu.VMEM((_BPW, _D), jnp.float32),
        pltpu.VMEM((_BPW, _D), jnp.float32),
        pltpu.VMEM((_BPW, _D), jnp.float32),
        pltpu.SemaphoreType.DMA,
        pltpu.SemaphoreType.DMA,
        pltpu.SemaphoreType.DMA,
    ],
    compiler_params=pltpu.CompilerParams(use_tc_tiling_on_sc=False),
)
def _lookup(h_hbm, t_hbm, r_hbm, ent_hbm, rel_hbm,
            h_out, t_out, r_out,
            hi_v, ti_v, ri_v, hr_v, tr_v, rr_v,
            sem_h, sem_t, sem_r):
    wid = lax.axis_index("s") * _NC + lax.axis_index("c")
    base = wid * _BPW
    sl = pl.ds(base, _BPW)
    pltpu.sync_copy(h_hbm.at[sl], hi_v)
    pltpu.sync_copy(t_hbm.at[sl], ti_v)
    pltpu.sync_copy(r_hbm.at[sl], ri_v)
    ch = pltpu.async_copy(ent_hbm.at[hi_v], hr_v, sem_h)
    ct = pltpu.async_copy(ent_hbm.at[ti_v], tr_v, sem_t)
    cr = pltpu.async_copy(rel_hbm.at[ri_v], rr_v, sem_r)
    ch.wait()
    pltpu.sync_copy(hr_v, h_out.at[sl])
    ct.wait()
    pltpu.sync_copy(tr_v, t_out.at[sl])
    cr.wait()
    pltpu.sync_copy(rr_v, r_out.at[sl])


def kernel(h, t, r, entity_table, relation_table):
    return _lookup(h.astype(jnp.int32), t.astype(jnp.int32),
                   r.astype(jnp.int32), entity_table, relation_table)


# speedup vs baseline: 1.0001x; 1.0001x over previous
import functools

import jax
import jax.numpy as jnp
from jax import lax
from jax.experimental import pallas as pl
from jax.experimental.pallas import tpu as pltpu, tpu_sc as plsc

_B = 16384
_D = 64

_NC = 2
_NS = 16
_NW = _NC * _NS
_BPW = _B // _NW

_mesh = plsc.VectorSubcoreMesh(core_axis_name="c", subcore_axis_name="s")


@functools.partial(
    pl.kernel,
    mesh=_mesh,
    out_type=(
        jax.ShapeDtypeStruct((_B, _D), jnp.float32),
        jax.ShapeDtypeStruct((_B, _D), jnp.float32),
        jax.ShapeDtypeStruct((_B, _D), jnp.float32),
    ),
    scratch_types=[
        pltpu.VMEM((_BPW,), jnp.int32),
        pltpu.VMEM((_BPW,), jnp.int32),
        pltpu.VMEM((_BPW,), jnp.int32),
        pltpu.VMEM((_BPW, _D), jnp.float32),
        pltpu.VMEM((_BPW, _D), jnp.float32),
        pltpu.VMEM((_BPW, _D), jnp.float32),
        pltpu.SemaphoreType.DMA,
        pltpu.SemaphoreType.DMA,
        pltpu.SemaphoreType.DMA,
        pltpu.SemaphoreType.DMA,
        pltpu.SemaphoreType.DMA,
        pltpu.SemaphoreType.DMA,
    ],
    compiler_params=pltpu.CompilerParams(use_tc_tiling_on_sc=False),
)
def _lookup(h_hbm, t_hbm, r_hbm, ent_hbm, rel_hbm,
            h_out, t_out, r_out,
            hi_v, ti_v, ri_v, hr_v, tr_v, rr_v,
            sem_h, sem_t, sem_r, sem_ho, sem_to, sem_ro):
    wid = lax.axis_index("s") * _NC + lax.axis_index("c")
    base = wid * _BPW
    sl = pl.ds(base, _BPW)
    ih = pltpu.async_copy(h_hbm.at[sl], hi_v, sem_ho)
    it = pltpu.async_copy(t_hbm.at[sl], ti_v, sem_to)
    ir = pltpu.async_copy(r_hbm.at[sl], ri_v, sem_ro)
    ih.wait()
    ch = pltpu.async_copy(ent_hbm.at[hi_v], hr_v, sem_h)
    it.wait()
    ct = pltpu.async_copy(ent_hbm.at[ti_v], tr_v, sem_t)
    ir.wait()
    cr = pltpu.async_copy(rel_hbm.at[ri_v], rr_v, sem_r)
    ch.wait()
    oh = pltpu.async_copy(hr_v, h_out.at[sl], sem_ho)
    ct.wait()
    ot = pltpu.async_copy(tr_v, t_out.at[sl], sem_to)
    cr.wait()
    orr = pltpu.async_copy(rr_v, r_out.at[sl], sem_ro)
    oh.wait()
    ot.wait()
    orr.wait()


def kernel(h, t, r, entity_table, relation_table):
    return _lookup(h.astype(jnp.int32), t.astype(jnp.int32),
                   r.astype(jnp.int32), entity_table, relation_table)
